# baseline (speedup 1.0000x reference)
import functools

import jax
import jax.numpy as jnp
from jax import lax
from jax.experimental import pallas as pl
from jax.experimental.pallas import tpu as pltpu
from jax.experimental.pallas import tpu_sc as plsc

BT = 256
NW = 32


def _route_body(x_ref, wr_ref, u_ref, pos0_ref, pos1_ref, g0_ref, g1_ref,
                gid_ref, act_ref, *, T, D, E, NB):
    x = x_ref[...]
    logits = lax.dot_general(
        x, wr_ref[...], (((1,), (1,)), ((), ())),
        precision=lax.Precision.DEFAULT,
        preferred_element_type=jnp.float32)
    iota = lax.broadcasted_iota(jnp.int32, (T, E), 1)
    i1 = jnp.argmax(logits, axis=1)[:, None]
    m1 = jnp.max(logits, axis=1, keepdims=True)
    l2 = jnp.where(iota == i1, -jnp.inf, logits)
    i2 = jnp.argmax(l2, axis=1)[:, None]
    m2 = jnp.max(l2, axis=1, keepdims=True)
    g0 = jax.nn.sigmoid(m1 - m2)
    g0_ref[...] = g0
    g1_ref[...] = 1.0 - g0

    z = 2.0 * x
    var = jnp.mean(z * z, axis=1, keepdims=True)
    u_ref[...] = z * lax.rsqrt(var + 1e-6)

    A = (iota == i1).astype(jnp.float32)
    B = (iota == i2).astype(jnp.float32)

    ti = lax.broadcasted_iota(jnp.int32, (T, T), 0)
    tj = lax.broadcasted_iota(jnp.int32, (T, T), 1)
    Ls = (ti > tj).astype(jnp.bfloat16)
    cumA = lax.dot_general(Ls, A.astype(jnp.bfloat16), (((1,), (0,)), ((), ())),
                           preferred_element_type=jnp.float32)
    cumB = lax.dot_general(Ls, B.astype(jnp.bfloat16), (((1,), (0,)), ((), ())),
                           preferred_element_type=jnp.float32)

    cnt1 = jnp.sum(A, axis=0, keepdims=True)
    cnt = cnt1 + jnp.sum(B, axis=0, keepdims=True)
    pc = jnp.ceil(cnt / BT) * BT
    ei = lax.broadcasted_iota(jnp.int32, (E, E), 0)
    ej = lax.broadcasted_iota(jnp.int32, (E, E), 1)
    Mlt = (ei < ej).astype(jnp.float32)
    start = lax.dot_general(pc, Mlt, (((1,), (0,)), ((), ())),
                            precision=lax.Precision.HIGHEST,
                            preferred_element_type=jnp.float32)

    pos0 = jnp.sum(A * (start + cumA), axis=1, keepdims=True)
    pos1 = jnp.sum(B * (start + cnt1 + cumB), axis=1, keepdims=True)
    pos0_ref[...] = pos0.astype(jnp.int32)
    pos1_ref[...] = pos1.astype(jnp.int32)

    I8 = (ei == ej).astype(jnp.float32)
    start_c = lax.dot_general(I8, start, (((1,), (1,)), ((), ())),
                              precision=lax.Precision.HIGHEST,
                              preferred_element_type=jnp.float32)
    cnt_c = lax.dot_general(I8, cnt, (((1,), (1,)), ((), ())),
                            precision=lax.Precision.HIGHEST,
                            preferred_element_type=jnp.float32)
    bBT = (lax.broadcasted_iota(jnp.int32, (E, 128), 1) * BT).astype(jnp.float32)
    ge = (bBT >= start_c).astype(jnp.float32)
    gid = jnp.sum(ge, axis=0, keepdims=True) - 1.0
    act = jnp.sum(ge * (bBT < start_c + cnt_c).astype(jnp.float32),
                  axis=0, keepdims=True)
    gid_ref[...] = jnp.maximum(gid, 0.0).astype(jnp.int32)
    act_ref[...] = act.astype(jnp.int32)


def _route_call(T, D, E, NB, interpret=False):
    body = functools.partial(_route_body, T=T, D=D, E=E, NB=NB)
    f32, i32 = jnp.float32, jnp.int32
    return pl.pallas_call(
        body,
        in_specs=[pl.BlockSpec((T, D), lambda: (0, 0)),
                  pl.BlockSpec((E, D), lambda: (0, 0))],
        out_specs=[pl.BlockSpec((T, D), lambda: (0, 0)),
                   pl.BlockSpec((T, 1), lambda: (0, 0)),
                   pl.BlockSpec((T, 1), lambda: (0, 0)),
                   pl.BlockSpec((T, 1), lambda: (0, 0)),
                   pl.BlockSpec((T, 1), lambda: (0, 0)),
                   pl.BlockSpec((1, 128), lambda: (0, 0)),
                   pl.BlockSpec((1, 128), lambda: (0, 0))],
        out_shape=[jax.ShapeDtypeStruct((T, D), f32),
                   jax.ShapeDtypeStruct((T, 1), i32),
                   jax.ShapeDtypeStruct((T, 1), i32),
                   jax.ShapeDtypeStruct((T, 1), f32),
                   jax.ShapeDtypeStruct((T, 1), f32),
                   jax.ShapeDtypeStruct((1, 128), i32),
                   jax.ShapeDtypeStruct((1, 128), i32)],
        interpret=interpret,
    )


def _dispatch_sc(u, pos0, pos1, P):
    T, D = u.shape
    TPW = T // NW
    mesh = plsc.VectorSubcoreMesh(core_axis_name="c", subcore_axis_name="s")

    @functools.partial(
        pl.kernel, mesh=mesh,
        out_type=jax.ShapeDtypeStruct((P, D), jnp.float32),
        scratch_types=[pltpu.VMEM((TPW,), jnp.int32),
                       pltpu.VMEM((TPW,), jnp.int32),
                       pltpu.VMEM((TPW, D), jnp.float32),
                       pltpu.SemaphoreType.DMA],
    )
    def k2(u_hbm, p0_hbm, p1_hbm, xs_hbm, idx0_v, idx1_v, rows_v, sem):
        wid = lax.axis_index("s") * 2 + lax.axis_index("c")
        base = wid * TPW
        pltpu.sync_copy(p0_hbm.at[pl.ds(base, TPW)], idx0_v)
        pltpu.sync_copy(p1_hbm.at[pl.ds(base, TPW)], idx1_v)
        pltpu.sync_copy(u_hbm.at[pl.ds(base, TPW)], rows_v)
        pltpu.async_copy(rows_v, xs_hbm.at[idx0_v], sem).wait()
        pltpu.async_copy(rows_v, xs_hbm.at[idx1_v], sem).wait()

    return k2(u, pos0, pos1)


def _group_body(gid_ref, act_ref, xs_ref, w13g_ref, w13u_ref, w2_ref, g_ref,
                o_ref, zb, *, HC):
    b = pl.program_id(0)
    hc = pl.program_id(1)

    @pl.when(act_ref[b] == 1)
    def _():
        @pl.when(hc == 0)
        def _():
            zb[...] = (xs_ref[...] * g_ref[0]).astype(jnp.bfloat16)

        hg = lax.dot_general(zb[...], w13g_ref[0].astype(jnp.bfloat16),
                             (((1,), (1,)), ((), ())),
                             preferred_element_type=jnp.float32)
        hu = lax.dot_general(zb[...], w13u_ref[0].astype(jnp.bfloat16),
                             (((1,), (1,)), ((), ())),
                             preferred_element_type=jnp.float32)
        sw = (hg * jax.nn.sigmoid(hg) * hu).astype(jnp.bfloat16)
        proj = lax.dot_general(sw, w2_ref[0].astype(jnp.bfloat16),
                               (((1,), (1,)), ((), ())),
                               preferred_element_type=jnp.float32)

        @pl.when(hc == 0)
        def _():
            o_ref[...] = proj

        @pl.when(hc != 0)
        def _():
            o_ref[...] += proj


def _group_call(D, E, H, NB, HC, interpret=False):
    CH = H // HC
    body = functools.partial(_group_body, HC=HC)
    grid_spec = pltpu.PrefetchScalarGridSpec(
        num_scalar_prefetch=2,
        grid=(NB, HC),
        in_specs=[
            pl.BlockSpec((BT, D), lambda b, hc, gid, act: (b, 0)),
            pl.BlockSpec((1, CH, D),
                         lambda b, hc, gid, act: (gid[b], hc, 0)),
            pl.BlockSpec((1, CH, D),
                         lambda b, hc, gid, act: (gid[b], hc + HC, 0)),
            pl.BlockSpec((1, D, CH),
                         lambda b, hc, gid, act: (gid[b], 0, hc)),
            pl.BlockSpec((1, 1, D), lambda b, hc, gid, act: (gid[b], 0, 0)),
        ],
        out_specs=pl.BlockSpec((BT, D), lambda b, hc, gid, act: (b, 0)),
        scratch_shapes=[pltpu.VMEM((BT, D), jnp.bfloat16)],
    )
    return pl.pallas_call(
        body,
        grid_spec=grid_spec,
        out_shape=jax.ShapeDtypeStruct((NB * BT, D), jnp.float32),
        compiler_params=pltpu.CompilerParams(
            dimension_semantics=("arbitrary", "arbitrary")),
        interpret=interpret,
    )


def _collect_sc(outs, pos0, pos1, T):
    P, D = outs.shape
    TPW = T // NW
    mesh = plsc.VectorSubcoreMesh(core_axis_name="c", subcore_axis_name="s")
    f32 = jnp.float32

    @functools.partial(
        pl.kernel, mesh=mesh,
        out_type=(jax.ShapeDtypeStruct((T, D), f32),
                  jax.ShapeDtypeStruct((T, D), f32)),
        scratch_types=[pltpu.VMEM((TPW,), jnp.int32),
                       pltpu.VMEM((TPW,), jnp.int32),
                       pltpu.VMEM((TPW, D), f32),
                       pltpu.VMEM((TPW, D), f32),
                       pltpu.SemaphoreType.DMA],
    )
    def k4(o_hbm, p0_hbm, p1_hbm, a0_hbm, a1_hbm,
           idx0_v, idx1_v, r0_v, r1_v, sem):
        wid = lax.axis_index("s") * 2 + lax.axis_index("c")
        base = wid * TPW
        pltpu.sync_copy(p0_hbm.at[pl.ds(base, TPW)], idx0_v)
        pltpu.sync_copy(p1_hbm.at[pl.ds(base, TPW)], idx1_v)
        pltpu.async_copy(o_hbm.at[idx0_v], r0_v, sem).wait()
        pltpu.async_copy(o_hbm.at[idx1_v], r1_v, sem).wait()
        pltpu.sync_copy(r0_v, a0_hbm.at[pl.ds(base, TPW)])
        pltpu.sync_copy(r1_v, a1_hbm.at[pl.ds(base, TPW)])

    return k4(outs, pos0, pos1)


def _combine_body(x_ref, a0_ref, a1_ref, g0_ref, g1_ref, o_ref):
    o_ref[...] = (x_ref[...] + g0_ref[...] * a0_ref[...]
                  + g1_ref[...] * a1_ref[...])


def _combine_call(T, D, interpret=False):
    return pl.pallas_call(
        _combine_body,
        in_specs=[pl.BlockSpec((T, D), lambda: (0, 0)),
                  pl.BlockSpec((T, D), lambda: (0, 0)),
                  pl.BlockSpec((T, D), lambda: (0, 0)),
                  pl.BlockSpec((T, 1), lambda: (0, 0)),
                  pl.BlockSpec((T, 1), lambda: (0, 0))],
        out_specs=pl.BlockSpec((T, D), lambda: (0, 0)),
        out_shape=jax.ShapeDtypeStruct((T, D), jnp.float32),
        interpret=interpret,
    )


def kernel(x, Wr, W13, W2, g_norm):
    T, D = x.shape
    E = Wr.shape[0]
    H = W2.shape[2]
    NB = (2 * T) // BT + E
    P = NB * BT

    u, pos0, pos1, g0, g1, gid_row, act_row = _route_call(T, D, E, NB)(x, Wr)
    pos0f = pos0.reshape(T)
    pos1f = pos1.reshape(T)
    gid = gid_row.reshape(128)[:NB]
    act = act_row.reshape(128)[:NB]

    xs = _dispatch_sc(u, pos0f, pos1f, P)
    outs = _group_call(D, E, H, NB, HC=2)(
        gid, act, xs, W13, W13, W2, g_norm.reshape(E, 1, D))
    a0, a1 = _collect_sc(outs, pos0f, pos1f, T)
    return _combine_call(T, D)(x, a0, a1, g0, g1)


# speedup vs baseline: 1.3214x; 1.3214x over previous
import functools

import jax
import jax.numpy as jnp
from jax import lax
from jax.experimental import pallas as pl
from jax.experimental.pallas import tpu as pltpu
from jax.experimental.pallas import tpu_sc as plsc

BT = 256
NW = 32


def _route_body(x_ref, wr_ref, u_ref, pos0_ref, pos1_ref, g0_ref, g1_ref,
                gid_ref, act_ref, *, T, D, E, NB):
    x = x_ref[...]
    logits = lax.dot_general(
        x, wr_ref[...], (((1,), (1,)), ((), ())),
        precision=lax.Precision.DEFAULT,
        preferred_element_type=jnp.float32)
    iota = lax.broadcasted_iota(jnp.int32, (T, E), 1)
    i1 = jnp.argmax(logits, axis=1)[:, None]
    m1 = jnp.max(logits, axis=1, keepdims=True)
    l2 = jnp.where(iota == i1, -jnp.inf, logits)
    i2 = jnp.argmax(l2, axis=1)[:, None]
    m2 = jnp.max(l2, axis=1, keepdims=True)
    g0 = jax.nn.sigmoid(m1 - m2)
    g0_ref[...] = g0
    g1_ref[...] = 1.0 - g0

    z = 2.0 * x
    var = jnp.mean(z * z, axis=1, keepdims=True)
    u_ref[...] = z * lax.rsqrt(var + 1e-6)

    A = (iota == i1).astype(jnp.float32)
    B = (iota == i2).astype(jnp.float32)

    GT = 256
    G = T // GT
    AB = jnp.concatenate([A, B], axis=1).astype(jnp.bfloat16)
    gi = lax.broadcasted_iota(jnp.int32, (GT, GT), 0)
    gj = lax.broadcasted_iota(jnp.int32, (GT, GT), 1)
    Lg = (gi > gj).astype(jnp.bfloat16)
    cums = []
    tots = []
    for g in range(G):
        ABg = AB[g * GT:(g + 1) * GT]
        cums.append(lax.dot_general(Lg, ABg, (((1,), (0,)), ((), ())),
                                    preferred_element_type=jnp.float32))
        tots.append(jnp.sum(ABg.astype(jnp.float32), axis=0, keepdims=True))
    totals = jnp.concatenate(tots, axis=0)
    hi = lax.broadcasted_iota(jnp.int32, (G, G), 0)
    hj = lax.broadcasted_iota(jnp.int32, (G, G), 1)
    Lh = (hi > hj).astype(jnp.float32)
    offs = lax.dot_general(Lh, totals, (((1,), (0,)), ((), ())),
                           precision=lax.Precision.HIGHEST,
                           preferred_element_type=jnp.float32)
    cumAB = jnp.concatenate(
        [cums[g] + offs[g:g + 1] for g in range(G)], axis=0)
    cumA = cumAB[:, :E]
    cumB = cumAB[:, E:]

    cnt1 = jnp.sum(A, axis=0, keepdims=True)
    cnt = cnt1 + jnp.sum(B, axis=0, keepdims=True)
    pc = jnp.ceil(cnt / BT) * BT
    ei = lax.broadcasted_iota(jnp.int32, (E, E), 0)
    ej = lax.broadcasted_iota(jnp.int32, (E, E), 1)
    Mlt = (ei < ej).astype(jnp.float32)
    start = lax.dot_general(pc, Mlt, (((1,), (0,)), ((), ())),
                            precision=lax.Precision.HIGHEST,
                            preferred_element_type=jnp.float32)

    pos0 = jnp.sum(A * (start + cumA), axis=1, keepdims=True)
    pos1 = jnp.sum(B * (start + cnt1 + cumB), axis=1, keepdims=True)
    pos0_ref[...] = pos0.astype(jnp.int32)
    pos1_ref[...] = pos1.astype(jnp.int32)

    I8 = (ei == ej).astype(jnp.float32)
    start_c = lax.dot_general(I8, start, (((1,), (1,)), ((), ())),
                              precision=lax.Precision.HIGHEST,
                              preferred_element_type=jnp.float32)
    cnt_c = lax.dot_general(I8, cnt, (((1,), (1,)), ((), ())),
                            precision=lax.Precision.HIGHEST,
                            preferred_element_type=jnp.float32)
    bBT = (lax.broadcasted_iota(jnp.int32, (E, 128), 1) * BT).astype(jnp.float32)
    ge = (bBT >= start_c).astype(jnp.float32)
    gid = jnp.sum(ge, axis=0, keepdims=True) - 1.0
    act = jnp.sum(ge * (bBT < start_c + cnt_c).astype(jnp.float32),
                  axis=0, keepdims=True)
    eio = lax.broadcasted_iota(jnp.int32, (1, E), 1).astype(jnp.float32)
    hav = jnp.max(jnp.where(cnt > 0, eio, 0.0))
    gid_ref[...] = jnp.clip(gid, 0.0, hav).astype(jnp.int32)
    act_ref[...] = act.astype(jnp.int32)


def _route_call(T, D, E, NB, interpret=False):
    body = functools.partial(_route_body, T=T, D=D, E=E, NB=NB)
    f32, i32 = jnp.float32, jnp.int32
    return pl.pallas_call(
        body,
        in_specs=[pl.BlockSpec((T, D), lambda: (0, 0)),
                  pl.BlockSpec((E, D), lambda: (0, 0))],
        out_specs=[pl.BlockSpec((T, D), lambda: (0, 0)),
                   pl.BlockSpec((T, 1), lambda: (0, 0)),
                   pl.BlockSpec((T, 1), lambda: (0, 0)),
                   pl.BlockSpec((T, 1), lambda: (0, 0)),
                   pl.BlockSpec((T, 1), lambda: (0, 0)),
                   pl.BlockSpec((1, 128), lambda: (0, 0)),
                   pl.BlockSpec((1, 128), lambda: (0, 0))],
        out_shape=[jax.ShapeDtypeStruct((T, D), f32),
                   jax.ShapeDtypeStruct((T, 1), i32),
                   jax.ShapeDtypeStruct((T, 1), i32),
                   jax.ShapeDtypeStruct((T, 1), f32),
                   jax.ShapeDtypeStruct((T, 1), f32),
                   jax.ShapeDtypeStruct((1, 128), i32),
                   jax.ShapeDtypeStruct((1, 128), i32)],
        interpret=interpret,
    )


def _dispatch_sc(u, pos0, pos1, P):
    T, D = u.shape
    TPW = T // NW
    mesh = plsc.VectorSubcoreMesh(core_axis_name="c", subcore_axis_name="s")

    @functools.partial(
        pl.kernel, mesh=mesh,
        out_type=jax.ShapeDtypeStruct((P, D), jnp.float32),
        scratch_types=[pltpu.VMEM((TPW,), jnp.int32),
                       pltpu.VMEM((TPW,), jnp.int32),
                       pltpu.VMEM((TPW, D), jnp.float32),
                       pltpu.SemaphoreType.DMA],
    )
    def k2(u_hbm, p0_hbm, p1_hbm, xs_hbm, idx0_v, idx1_v, rows_v, sem):
        wid = lax.axis_index("s") * 2 + lax.axis_index("c")
        base = wid * TPW
        pltpu.sync_copy(p0_hbm.at[pl.ds(base, TPW)], idx0_v)
        pltpu.sync_copy(p1_hbm.at[pl.ds(base, TPW)], idx1_v)
        pltpu.sync_copy(u_hbm.at[pl.ds(base, TPW)], rows_v)
        pltpu.async_copy(rows_v, xs_hbm.at[idx0_v], sem).wait()
        pltpu.async_copy(rows_v, xs_hbm.at[idx1_v], sem).wait()

    return k2(u, pos0, pos1)


def _group_body(gid_ref, act_ref, xs_ref, w13g_ref, w13u_ref, w2_ref, g_ref,
                o_ref):
    b = pl.program_id(0)

    @pl.when(act_ref[b] == 1)
    def _():
        zb = (xs_ref[...] * g_ref[0]).astype(jnp.bfloat16)
        hg = lax.dot_general(zb, w13g_ref[0].astype(jnp.bfloat16),
                             (((1,), (1,)), ((), ())),
                             preferred_element_type=jnp.float32)
        hu = lax.dot_general(zb, w13u_ref[0].astype(jnp.bfloat16),
                             (((1,), (1,)), ((), ())),
                             preferred_element_type=jnp.float32)
        sw = (hg * jax.nn.sigmoid(hg) * hu).astype(jnp.bfloat16)
        o_ref[...] = lax.dot_general(sw, w2_ref[0].astype(jnp.bfloat16),
                                     (((1,), (1,)), ((), ())),
                                     preferred_element_type=jnp.float32)


def _group_call(D, E, H, NB, interpret=False):
    grid_spec = pltpu.PrefetchScalarGridSpec(
        num_scalar_prefetch=2,
        grid=(NB,),
        in_specs=[
            pl.BlockSpec((BT, D), lambda b, gid, act: (b, 0)),
            pl.BlockSpec((1, H, D), lambda b, gid, act: (gid[b], 0, 0)),
            pl.BlockSpec((1, H, D), lambda b, gid, act: (gid[b], 1, 0)),
            pl.BlockSpec((1, D, H), lambda b, gid, act: (gid[b], 0, 0)),
            pl.BlockSpec((1, 1, D), lambda b, gid, act: (gid[b], 0, 0)),
        ],
        out_specs=pl.BlockSpec((BT, D), lambda b, gid, act: (b, 0)),
        scratch_shapes=[],
    )
    return pl.pallas_call(
        _group_body,
        grid_spec=grid_spec,
        out_shape=jax.ShapeDtypeStruct((NB * BT, D), jnp.float32),
        compiler_params=pltpu.CompilerParams(
            dimension_semantics=("arbitrary",)),
        interpret=interpret,
    )


def _collect_sc(outs, pos0, pos1, T):
    P, D = outs.shape
    TPW = T // NW
    mesh = plsc.VectorSubcoreMesh(core_axis_name="c", subcore_axis_name="s")
    f32 = jnp.float32

    @functools.partial(
        pl.kernel, mesh=mesh,
        out_type=(jax.ShapeDtypeStruct((T, D), f32),
                  jax.ShapeDtypeStruct((T, D), f32)),
        scratch_types=[pltpu.VMEM((TPW,), jnp.int32),
                       pltpu.VMEM((TPW,), jnp.int32),
                       pltpu.VMEM((TPW, D), f32),
                       pltpu.VMEM((TPW, D), f32),
                       pltpu.SemaphoreType.DMA],
    )
    def k4(o_hbm, p0_hbm, p1_hbm, a0_hbm, a1_hbm,
           idx0_v, idx1_v, r0_v, r1_v, sem):
        wid = lax.axis_index("s") * 2 + lax.axis_index("c")
        base = wid * TPW
        pltpu.sync_copy(p0_hbm.at[pl.ds(base, TPW)], idx0_v)
        pltpu.sync_copy(p1_hbm.at[pl.ds(base, TPW)], idx1_v)
        pltpu.async_copy(o_hbm.at[idx0_v], r0_v, sem).wait()
        pltpu.async_copy(o_hbm.at[idx1_v], r1_v, sem).wait()
        pltpu.sync_copy(r0_v, a0_hbm.at[pl.ds(base, TPW)])
        pltpu.sync_copy(r1_v, a1_hbm.at[pl.ds(base, TPW)])

    return k4(outs, pos0, pos1)


def _combine_body(x_ref, a0_ref, a1_ref, g0_ref, g1_ref, o_ref):
    o_ref[...] = (x_ref[...] + g0_ref[...] * a0_ref[...]
                  + g1_ref[...] * a1_ref[...])


def _combine_call(T, D, interpret=False):
    return pl.pallas_call(
        _combine_body,
        in_specs=[pl.BlockSpec((T, D), lambda: (0, 0)),
                  pl.BlockSpec((T, D), lambda: (0, 0)),
                  pl.BlockSpec((T, D), lambda: (0, 0)),
                  pl.BlockSpec((T, 1), lambda: (0, 0)),
                  pl.BlockSpec((T, 1), lambda: (0, 0))],
        out_specs=pl.BlockSpec((T, D), lambda: (0, 0)),
        out_shape=jax.ShapeDtypeStruct((T, D), jnp.float32),
        interpret=interpret,
    )


def kernel(x, Wr, W13, W2, g_norm):
    T, D = x.shape
    E = Wr.shape[0]
    H = W2.shape[2]
    NB = (2 * T) // BT + E
    P = NB * BT

    u, pos0, pos1, g0, g1, gid_row, act_row = _route_call(T, D, E, NB)(x, Wr)
    pos0f = pos0.reshape(T)
    pos1f = pos1.reshape(T)
    gid = gid_row.reshape(128)[:NB]
    act = act_row.reshape(128)[:NB]

    xs = _dispatch_sc(u, pos0f, pos1f, P)
    outs = _group_call(D, E, H, NB)(
        gid, act, xs, W13, W13, W2, g_norm.reshape(E, 1, D))
    a0, a1 = _collect_sc(outs, pos0f, pos1f, T)
    return _combine_call(T, D)(x, a0, a1, g0, g1)


# speedup vs baseline: 1.3215x; 1.0001x over previous
import functools

import jax
import jax.numpy as jnp
from jax import lax
from jax.experimental import pallas as pl
from jax.experimental.pallas import tpu as pltpu
from jax.experimental.pallas import tpu_sc as plsc

BT = 256
NW = 32


def _route_body(x_ref, wr_ref, u_ref, pos0_ref, pos1_ref, g0_ref, g1_ref,
                gid_ref, act_ref, *, T, D, E, NB):
    x = x_ref[...]
    logits = lax.dot_general(
        x, wr_ref[...], (((1,), (1,)), ((), ())),
        precision=lax.Precision.DEFAULT,
        preferred_element_type=jnp.float32)
    iota = lax.broadcasted_iota(jnp.int32, (T, E), 1)
    i1 = jnp.argmax(logits, axis=1)[:, None]
    m1 = jnp.max(logits, axis=1, keepdims=True)
    l2 = jnp.where(iota == i1, -jnp.inf, logits)
    i2 = jnp.argmax(l2, axis=1)[:, None]
    m2 = jnp.max(l2, axis=1, keepdims=True)
    g0 = jax.nn.sigmoid(m1 - m2)
    g0_ref[...] = jnp.broadcast_to(g0, (T, 16))
    g1_ref[...] = jnp.broadcast_to(1.0 - g0, (T, 16))

    z = 2.0 * x
    var = jnp.mean(z * z, axis=1, keepdims=True)
    u_ref[...] = z * lax.rsqrt(var + 1e-6)

    A = (iota == i1).astype(jnp.float32)
    B = (iota == i2).astype(jnp.float32)

    GT = 256
    G = T // GT
    AB = jnp.concatenate([A, B], axis=1).astype(jnp.bfloat16)
    gi = lax.broadcasted_iota(jnp.int32, (GT, GT), 0)
    gj = lax.broadcasted_iota(jnp.int32, (GT, GT), 1)
    Lg = (gi > gj).astype(jnp.bfloat16)
    cums = []
    tots = []
    for g in range(G):
        ABg = AB[g * GT:(g + 1) * GT]
        cums.append(lax.dot_general(Lg, ABg, (((1,), (0,)), ((), ())),
                                    preferred_element_type=jnp.float32))
        tots.append(jnp.sum(ABg.astype(jnp.float32), axis=0, keepdims=True))
    totals = jnp.concatenate(tots, axis=0)
    hi = lax.broadcasted_iota(jnp.int32, (G, G), 0)
    hj = lax.broadcasted_iota(jnp.int32, (G, G), 1)
    Lh = (hi > hj).astype(jnp.float32)
    offs = lax.dot_general(Lh, totals, (((1,), (0,)), ((), ())),
                           precision=lax.Precision.HIGHEST,
                           preferred_element_type=jnp.float32)
    cumAB = jnp.concatenate(
        [cums[g] + offs[g:g + 1] for g in range(G)], axis=0)
    cumA = cumAB[:, :E]
    cumB = cumAB[:, E:]

    cnt1 = jnp.sum(A, axis=0, keepdims=True)
    cnt = cnt1 + jnp.sum(B, axis=0, keepdims=True)
    pc = jnp.ceil(cnt / BT) * BT
    ei = lax.broadcasted_iota(jnp.int32, (E, E), 0)
    ej = lax.broadcasted_iota(jnp.int32, (E, E), 1)
    Mlt = (ei < ej).astype(jnp.float32)
    start = lax.dot_general(pc, Mlt, (((1,), (0,)), ((), ())),
                            precision=lax.Precision.HIGHEST,
                            preferred_element_type=jnp.float32)

    pos0 = jnp.sum(A * (start + cumA), axis=1, keepdims=True)
    pos1 = jnp.sum(B * (start + cnt1 + cumB), axis=1, keepdims=True)
    pos0_ref[...] = pos0.astype(jnp.int32)
    pos1_ref[...] = pos1.astype(jnp.int32)

    I8 = (ei == ej).astype(jnp.float32)
    start_c = lax.dot_general(I8, start, (((1,), (1,)), ((), ())),
                              precision=lax.Precision.HIGHEST,
                              preferred_element_type=jnp.float32)
    cnt_c = lax.dot_general(I8, cnt, (((1,), (1,)), ((), ())),
                            precision=lax.Precision.HIGHEST,
                            preferred_element_type=jnp.float32)
    bBT = (lax.broadcasted_iota(jnp.int32, (E, 128), 1) * BT).astype(jnp.float32)
    ge = (bBT >= start_c).astype(jnp.float32)
    gid = jnp.sum(ge, axis=0, keepdims=True) - 1.0
    act = jnp.sum(ge * (bBT < start_c + cnt_c).astype(jnp.float32),
                  axis=0, keepdims=True)
    eio = lax.broadcasted_iota(jnp.int32, (1, E), 1).astype(jnp.float32)
    hav = jnp.max(jnp.where(cnt > 0, eio, 0.0))
    gid_ref[...] = jnp.clip(gid, 0.0, hav).astype(jnp.int32)
    act_ref[...] = act.astype(jnp.int32)


def _route_call(T, D, E, NB, interpret=False):
    body = functools.partial(_route_body, T=T, D=D, E=E, NB=NB)
    f32, i32 = jnp.float32, jnp.int32
    return pl.pallas_call(
        body,
        in_specs=[pl.BlockSpec((T, D), lambda: (0, 0)),
                  pl.BlockSpec((E, D), lambda: (0, 0))],
        out_specs=[pl.BlockSpec((T, D), lambda: (0, 0)),
                   pl.BlockSpec((T, 1), lambda: (0, 0)),
                   pl.BlockSpec((T, 1), lambda: (0, 0)),
                   pl.BlockSpec((T, 16), lambda: (0, 0)),
                   pl.BlockSpec((T, 16), lambda: (0, 0)),
                   pl.BlockSpec((1, 128), lambda: (0, 0)),
                   pl.BlockSpec((1, 128), lambda: (0, 0))],
        out_shape=[jax.ShapeDtypeStruct((T, D), f32),
                   jax.ShapeDtypeStruct((T, 1), i32),
                   jax.ShapeDtypeStruct((T, 1), i32),
                   jax.ShapeDtypeStruct((T, 16), f32),
                   jax.ShapeDtypeStruct((T, 16), f32),
                   jax.ShapeDtypeStruct((1, 128), i32),
                   jax.ShapeDtypeStruct((1, 128), i32)],
        interpret=interpret,
    )


def _dispatch_sc(u, pos0, pos1, P):
    T, D = u.shape
    TPW = T // NW
    mesh = plsc.VectorSubcoreMesh(core_axis_name="c", subcore_axis_name="s")

    @functools.partial(
        pl.kernel, mesh=mesh,
        out_type=jax.ShapeDtypeStruct((P, D), jnp.float32),
        scratch_types=[pltpu.VMEM((TPW,), jnp.int32),
                       pltpu.VMEM((TPW,), jnp.int32),
                       pltpu.VMEM((TPW, D), jnp.float32),
                       pltpu.SemaphoreType.DMA],
    )
    def k2(u_hbm, p0_hbm, p1_hbm, xs_hbm, idx0_v, idx1_v, rows_v, sem):
        wid = lax.axis_index("s") * 2 + lax.axis_index("c")
        base = wid * TPW
        pltpu.sync_copy(p0_hbm.at[pl.ds(base, TPW)], idx0_v)
        pltpu.sync_copy(p1_hbm.at[pl.ds(base, TPW)], idx1_v)
        pltpu.sync_copy(u_hbm.at[pl.ds(base, TPW)], rows_v)
        pltpu.async_copy(rows_v, xs_hbm.at[idx0_v], sem).wait()
        pltpu.async_copy(rows_v, xs_hbm.at[idx1_v], sem).wait()

    return k2(u, pos0, pos1)


def _group_body(gid_ref, act_ref, xs_ref, w13g_ref, w13u_ref, w2_ref, g_ref,
                o_ref):
    b = pl.program_id(0)

    @pl.when(act_ref[b] == 1)
    def _():
        zb = (xs_ref[...] * g_ref[0]).astype(jnp.bfloat16)
        hg = lax.dot_general(zb, w13g_ref[0].astype(jnp.bfloat16),
                             (((1,), (1,)), ((), ())),
                             preferred_element_type=jnp.float32)
        hu = lax.dot_general(zb, w13u_ref[0].astype(jnp.bfloat16),
                             (((1,), (1,)), ((), ())),
                             preferred_element_type=jnp.float32)
        sw = (hg * jax.nn.sigmoid(hg) * hu).astype(jnp.bfloat16)
        o_ref[...] = lax.dot_general(sw, w2_ref[0].astype(jnp.bfloat16),
                                     (((1,), (1,)), ((), ())),
                                     preferred_element_type=jnp.float32)


def _group_call(D, E, H, NB, interpret=False):
    grid_spec = pltpu.PrefetchScalarGridSpec(
        num_scalar_prefetch=2,
        grid=(NB,),
        in_specs=[
            pl.BlockSpec((BT, D), lambda b, gid, act: (b, 0)),
            pl.BlockSpec((1, H, D), lambda b, gid, act: (gid[b], 0, 0)),
            pl.BlockSpec((1, H, D), lambda b, gid, act: (gid[b], 1, 0)),
            pl.BlockSpec((1, D, H), lambda b, gid, act: (gid[b], 0, 0)),
            pl.BlockSpec((1, 1, D), lambda b, gid, act: (gid[b], 0, 0)),
        ],
        out_specs=pl.BlockSpec((BT, D), lambda b, gid, act: (b, 0)),
        scratch_shapes=[],
    )
    return pl.pallas_call(
        _group_body,
        grid_spec=grid_spec,
        out_shape=jax.ShapeDtypeStruct((NB * BT, D), jnp.float32),
        compiler_params=pltpu.CompilerParams(
            dimension_semantics=("arbitrary",)),
        interpret=interpret,
    )


def _collect_sc(outs, pos0, pos1, x, g0b, g1b):
    P, D = outs.shape
    T = x.shape[0]
    TPW = T // NW
    DV = D // 16
    mesh = plsc.VectorSubcoreMesh(core_axis_name="c", subcore_axis_name="s")
    f32 = jnp.float32

    @functools.partial(
        pl.kernel, mesh=mesh,
        out_type=jax.ShapeDtypeStruct((T, D), f32),
        scratch_types=[pltpu.VMEM((TPW,), jnp.int32),
                       pltpu.VMEM((TPW,), jnp.int32),
                       pltpu.VMEM((TPW, D), f32),
                       pltpu.VMEM((TPW, D), f32),
                       pltpu.VMEM((TPW, 16), f32),
                       pltpu.VMEM((TPW, 16), f32),
                       pltpu.SemaphoreType.DMA],
    )
    def k4(o_hbm, p0_hbm, p1_hbm, x_hbm, g0_hbm, g1_hbm, y_hbm,
           idx0_v, idx1_v, acc_v, r_v, g0_v, g1_v, sem):
        wid = lax.axis_index("s") * 2 + lax.axis_index("c")
        base = wid * TPW
        pltpu.sync_copy(p0_hbm.at[pl.ds(base, TPW)], idx0_v)
        pltpu.sync_copy(p1_hbm.at[pl.ds(base, TPW)], idx1_v)
        pltpu.sync_copy(x_hbm.at[pl.ds(base, TPW)], acc_v)
        pltpu.sync_copy(g0_hbm.at[pl.ds(base, TPW)], g0_v)
        pltpu.sync_copy(g1_hbm.at[pl.ds(base, TPW)], g1_v)

        def accum(g_v):
            def row(r, _):
                gv = g_v[r]
                for j in range(DV):
                    sl = pl.ds(j * 16, 16)
                    acc_v[r, sl] += gv * r_v[r, sl]
                return _
            lax.fori_loop(0, TPW, row, 0)

        pltpu.async_copy(o_hbm.at[idx0_v], r_v, sem).wait()
        accum(g0_v)
        pltpu.async_copy(o_hbm.at[idx1_v], r_v, sem).wait()
        accum(g1_v)
        pltpu.sync_copy(acc_v, y_hbm.at[pl.ds(base, TPW)])

    return k4(outs, pos0, pos1, x, g0b, g1b)


def kernel(x, Wr, W13, W2, g_norm):
    T, D = x.shape
    E = Wr.shape[0]
    H = W2.shape[2]
    NB = (2 * T) // BT + E
    P = NB * BT

    u, pos0, pos1, g0b, g1b, gid_row, act_row = _route_call(T, D, E, NB)(x, Wr)
    pos0f = pos0.reshape(T)
    pos1f = pos1.reshape(T)
    gid = gid_row.reshape(128)[:NB]
    act = act_row.reshape(128)[:NB]

    xs = _dispatch_sc(u, pos0f, pos1f, P)
    outs = _group_call(D, E, H, NB)(
        gid, act, xs, W13, W13, W2, g_norm.reshape(E, 1, D))
    return _collect_sc(outs, pos0f, pos1f, x, g0b, g1b)
